# baseline (speedup 1.0000x reference)
import jax
import jax.numpy as jnp
from jax import lax
from jax.experimental import pallas as pl
from jax.experimental.pallas import tpu as pltpu
from jax.experimental.pallas import tpu_sc as plsc

_N = 10000
_E = 320000
_D = 128
_DP = 144
_NC = 2
_NS = 16
_NW = _NC * _NS
_EPW = _E // _NW
_K = 128
_NF = 78
_TAIL = _EPW - _NF * _K
_R632 = 632
_RLAST = _N - 15 * _R632


def _sc_body(xp_hbm, edge_hbm, z2_hbm, psum_hbm,
             src0, dst0, rows0, src1, dst1, rows1, srcT, dstT,
             acc,
             si0, sg0, ss0, si1, sg1, ss1):
    c = lax.axis_index("c")
    s = lax.axis_index("s")
    wid = s * _NC + c
    base0 = wid * _EPW

    pltpu.sync_copy(z2_hbm.at[pl.ds(0, _K)], rows0)

    @pl.when(s < 15)
    def _():
        for j in range(4):
            pltpu.sync_copy(rows0, acc.at[pl.ds(s * _R632 + j * _K, _K)])
        pltpu.sync_copy(rows0.at[pl.ds(0, 120)],
                        acc.at[pl.ds(s * _R632 + 4 * _K, 120)])

    @pl.when(s == 15)
    def _():
        for j in range(4):
            pltpu.sync_copy(rows0, acc.at[pl.ds(15 * _R632 + j * _K, _K)])
        pltpu.sync_copy(rows0.at[pl.ds(0, 8)],
                        acc.at[pl.ds(15 * _R632 + 4 * _K, 8)])

    plsc.subcore_barrier()

    bufs = ((src0, dst0, rows0, si0, sg0, ss0),
            (src1, dst1, rows1, si1, sg1, ss1))

    def start_idx(t, b):
        base = base0 + t * _K
        sb, db, _, sib, _, _ = bufs[b]
        pltpu.async_copy(edge_hbm.at[pl.ds(base, _K)], sb, sib)
        pltpu.async_copy(edge_hbm.at[pl.ds(_E + base, _K)], db, sib)

    def wait_idx(t, b):
        base = base0 + t * _K
        sb, db, _, sib, _, _ = bufs[b]
        pltpu.make_async_copy(edge_hbm.at[pl.ds(base, _K)], sb, sib).wait()
        pltpu.make_async_copy(edge_hbm.at[pl.ds(_E + base, _K)], db, sib).wait()

    def start_gather(b):
        sb, _, rb, _, sgb, _ = bufs[b]
        pltpu.async_copy(xp_hbm.at[sb], rb, sgb)

    def wait_gather(b):
        sb, _, rb, _, sgb, _ = bufs[b]
        pltpu.make_async_copy(xp_hbm.at[sb], rb, sgb).wait()

    def start_scatter(b):
        _, db, rb, _, _, ssb = bufs[b]
        pltpu.async_copy(rb, acc.at[db], ssb, add=True)

    def wait_scatter(b):
        _, db, rb, _, _, ssb = bufs[b]
        pltpu.make_async_copy(rb, acc.at[db], ssb).wait()

    def step(t, b, first):
        b1 = 1 - b
        if first:
            @pl.when(t > 0)
            def _():
                wait_scatter(b1)
        else:
            wait_scatter(b1)

        @pl.when(t + 1 < _NF)
        def _():
            start_idx(t + 1, b1)
        wait_gather(b)
        start_scatter(b)

        @pl.when(t + 1 < _NF)
        def _():
            wait_idx(t + 1, b1)
            start_gather(b1)

    start_idx(0, 0)
    wait_idx(0, 0)
    start_gather(0)

    def pair(i, carry):
        t0 = 2 * i
        step(t0, 0, True)
        step(t0 + 1, 1, False)
        return carry

    lax.fori_loop(0, _NF // 2, pair, 0)
    wait_scatter(1)

    tb = base0 + _NF * _K
    pltpu.sync_copy(edge_hbm.at[pl.ds(tb, _TAIL)], srcT)
    pltpu.sync_copy(edge_hbm.at[pl.ds(_E + tb, _TAIL)], dstT)
    pltpu.async_copy(xp_hbm.at[srcT], rows0.at[pl.ds(0, _TAIL)], sg0).wait()
    pltpu.sync_copy(rows0.at[pl.ds(0, _TAIL)], acc.at[dstT], add=True)

    plsc.subcore_barrier()

    @pl.when(s < 15)
    def _():
        for j in range(4):
            pltpu.sync_copy(acc.at[pl.ds(s * _R632 + j * _K, _K)], rows0)
            pltpu.sync_copy(
                rows0, psum_hbm.at[c, pl.ds(s * _R632 + j * _K, _K)])
        pltpu.sync_copy(acc.at[pl.ds(s * _R632 + 4 * _K, 120)],
                        rows0.at[pl.ds(0, 120)])
        pltpu.sync_copy(rows0.at[pl.ds(0, 120)],
                        psum_hbm.at[c, pl.ds(s * _R632 + 4 * _K, 120)])

    @pl.when(s == 15)
    def _():
        for j in range(4):
            pltpu.sync_copy(acc.at[pl.ds(15 * _R632 + j * _K, _K)], rows0)
            pltpu.sync_copy(
                rows0, psum_hbm.at[c, pl.ds(15 * _R632 + j * _K, _K)])
        pltpu.sync_copy(acc.at[pl.ds(15 * _R632 + 4 * _K, 8)],
                        rows0.at[pl.ds(0, 8)])
        pltpu.sync_copy(rows0.at[pl.ds(0, 8)],
                        psum_hbm.at[c, pl.ds(15 * _R632 + 4 * _K, 8)])


def _sc_segment_sum(x_pad, edge_index):
    mesh = plsc.VectorSubcoreMesh(core_axis_name="c", subcore_axis_name="s")
    z2 = jnp.zeros((_N, _DP), jnp.float32)
    run = pl.kernel(
        _sc_body,
        out_type=jax.ShapeDtypeStruct((_NC, _N, _DP), jnp.float32),
        mesh=mesh,
        scratch_types=[
            pltpu.VMEM((_K,), jnp.int32),
            pltpu.VMEM((_K,), jnp.int32),
            pltpu.VMEM((_K, _DP), jnp.float32),
            pltpu.VMEM((_K,), jnp.int32),
            pltpu.VMEM((_K,), jnp.int32),
            pltpu.VMEM((_K, _DP), jnp.float32),
            pltpu.VMEM((_TAIL,), jnp.int32),
            pltpu.VMEM((_TAIL,), jnp.int32),
            pltpu.VMEM_SHARED((_N, _DP), jnp.float32),
            pltpu.SemaphoreType.DMA,
            pltpu.SemaphoreType.DMA,
            pltpu.SemaphoreType.DMA,
            pltpu.SemaphoreType.DMA,
            pltpu.SemaphoreType.DMA,
            pltpu.SemaphoreType.DMA,
        ],
        compiler_params=pltpu.CompilerParams(use_tc_tiling_on_sc=False),
    )
    return run(x_pad, edge_index.reshape(-1), z2)


_R = 2000


def _tc_body(psum_ref, x_ref, wl_ref, bl_ref, wr_ref, w1_ref, w2_ref,
             out1_ref, out2_ref, x1_ref):
    summed = psum_ref[0] + psum_ref[1]
    cnt = summed[:, _D:_D + 1]
    mean = summed[:, :_D] / jnp.maximum(cnt, 1.0)
    x_blk = x_ref[...]
    x1 = (jnp.dot(mean, wl_ref[...].T, preferred_element_type=jnp.float32,
                  precision=lax.Precision.HIGHEST)
          + bl_ref[...]
          + jnp.dot(x_blk, wr_ref[...].T, preferred_element_type=jnp.float32,
                    precision=lax.Precision.HIGHEST))
    x1_ref[...] = x1
    norm = jnp.sqrt(jnp.sum(x1 * x1, axis=1, keepdims=True))
    xn = x1 / jnp.maximum(norm, 1e-12)
    w1 = w1_ref[...]
    w1 = w1 / jnp.maximum(jnp.sqrt(jnp.sum(w1 * w1, axis=0, keepdims=True)), 1e-12)
    out1_ref[...] = 10.0 * jnp.dot(xn, w1, preferred_element_type=jnp.float32,
                                   precision=lax.Precision.HIGHEST)
    w2 = w2_ref[...]
    w2 = w2 / jnp.maximum(jnp.sqrt(jnp.sum(w2 * w2, axis=0, keepdims=True)), 1e-12)
    out2_ref[...] = 10.0 * jnp.dot(xn, w2, preferred_element_type=jnp.float32,
                                   precision=lax.Precision.HIGHEST)


def _tc_heads(psum, x, W_l, b_l, W_r, W1, W2):
    c1 = W1.shape[1]
    c2 = W2.shape[1]
    grid = _N // _R
    return pl.pallas_call(
        _tc_body,
        grid=(grid,),
        in_specs=[
            pl.BlockSpec((_NC, _R, _DP), lambda i: (0, i, 0)),
            pl.BlockSpec((_R, _D), lambda i: (i, 0)),
            pl.BlockSpec((_D, _D), lambda i: (0, 0)),
            pl.BlockSpec((1, _D), lambda i: (0, 0)),
            pl.BlockSpec((_D, _D), lambda i: (0, 0)),
            pl.BlockSpec((_D, c1), lambda i: (0, 0)),
            pl.BlockSpec((_D, c2), lambda i: (0, 0)),
        ],
        out_specs=[
            pl.BlockSpec((_R, c1), lambda i: (i, 0)),
            pl.BlockSpec((_R, c2), lambda i: (i, 0)),
            pl.BlockSpec((_R, _D), lambda i: (i, 0)),
        ],
        out_shape=[
            jax.ShapeDtypeStruct((_N, c1), jnp.float32),
            jax.ShapeDtypeStruct((_N, c2), jnp.float32),
            jax.ShapeDtypeStruct((_N, _D), jnp.float32),
        ],
    )(psum, x, W_l, b_l, W_r, W1, W2)


@jax.jit
def kernel(x, edge_index, W_l, b_l, W_r, W1, W2):
    x_pad = jnp.concatenate(
        [x, jnp.ones((_N, 1), jnp.float32),
         jnp.zeros((_N, _DP - _D - 1), jnp.float32)], axis=1)
    psum = _sc_segment_sum(x_pad, edge_index)
    out1, out2, x1 = _tc_heads(psum, x, W_l, b_l[None, :], W_r, W1, W2)
    return (out1, out2, x1)


# speedup vs baseline: 1.3109x; 1.3109x over previous
import jax
import jax.numpy as jnp
from jax import lax
from jax.experimental import pallas as pl
from jax.experimental.pallas import tpu as pltpu
from jax.experimental.pallas import tpu_sc as plsc

_N = 10000
_E = 320000
_D = 128
_NC = 2
_NS = 16
_NW = _NC * _NS
_EPW = _E // _NW
_K = 128
_NF = 78
_TAIL = _EPW - _NF * _K
_R632 = 632
_RLAST = _N - 15 * _R632
_NP = 10112
_C640 = 640
_CLAST = _NP - 15 * _C640


def _sc_body(x_hbm, edge_hbm, z2_hbm, z1_hbm, psum_hbm, pcnt_hbm,
             src_all, dst0, dst1, dst2, rows0, rows1, dstT, ones_v, zcnt,
             acc, cntacc,
             si0, si1, si2, sg0, sg1, ss0, ss1):
    c = lax.axis_index("c")
    s = lax.axis_index("s")
    wid = s * _NC + c
    base0 = wid * _EPW

    pltpu.async_copy(edge_hbm.at[pl.ds(base0, _EPW)], src_all, si2)

    pltpu.sync_copy(z2_hbm.at[pl.ds(0, _K)], rows0)
    pltpu.sync_copy(z1_hbm.at[pl.ds(0, _C640)], zcnt)

    @pl.when(s < 15)
    def _():
        for j in range(4):
            pltpu.sync_copy(rows0, acc.at[pl.ds(s * _R632 + j * _K, _K)])
        pltpu.sync_copy(rows0.at[pl.ds(0, 120)],
                        acc.at[pl.ds(s * _R632 + 4 * _K, 120)])
        pltpu.sync_copy(zcnt, cntacc.at[pl.ds(s * _C640, _C640)])

    @pl.when(s == 15)
    def _():
        for j in range(4):
            pltpu.sync_copy(rows0, acc.at[pl.ds(15 * _R632 + j * _K, _K)])
        pltpu.sync_copy(rows0.at[pl.ds(0, 8)],
                        acc.at[pl.ds(15 * _R632 + 4 * _K, 8)])
        pltpu.sync_copy(zcnt.at[pl.ds(0, _CLAST)],
                        cntacc.at[pl.ds(15 * _C640, _CLAST)])

    for i in range(_K // 16):
        ones_v[pl.ds(i * 16, 16)] = jnp.ones((16,), jnp.float32)

    pltpu.make_async_copy(edge_hbm.at[pl.ds(base0, _EPW)], src_all, si2).wait()

    plsc.subcore_barrier()

    dsts = (dst0, dst1, dst2)
    isems = (si0, si1, si2)
    rows = (rows0, rows1)
    gsems = (sg0, sg1)
    ssems = (ss0, ss1)

    def start_idx(t, d):
        pltpu.async_copy(edge_hbm.at[pl.ds(_E + base0 + t * _K, _K)],
                         dsts[d], isems[d])

    def wait_idx(t, d):
        pltpu.make_async_copy(edge_hbm.at[pl.ds(_E + base0 + t * _K, _K)],
                              dsts[d], isems[d]).wait()

    def start_gather(t, r):
        pltpu.async_copy(x_hbm.at[src_all.at[pl.ds(t * _K, _K)]],
                         rows[r], gsems[r])

    def wait_gather(t, r):
        pltpu.make_async_copy(x_hbm.at[src_all.at[pl.ds(t * _K, _K)]],
                              rows[r], gsems[r]).wait()

    def start_scatter(r, d):
        pltpu.async_copy(rows[r], acc.at[dsts[d]], ssems[r], add=True)
        pltpu.async_copy(ones_v, cntacc.at[dsts[d]], ssems[r], add=True)

    def wait_scatter(r, d):
        pltpu.make_async_copy(rows[r], acc.at[dsts[d]], ssems[r]).wait()
        pltpu.make_async_copy(ones_v, cntacc.at[dsts[d]], ssems[r]).wait()

    def step(t, k):
        r = k % 2
        d = k % 3
        if k == 0:
            @pl.when(t > 0)
            def _():
                wait_scatter(1, 2)
        else:
            wait_scatter((k - 1) % 2, (k - 1) % 3)

        if k == 5:
            @pl.when(t + 1 < _NF)
            def _():
                wait_idx(t + 1, (k + 1) % 3)
                start_gather(t + 1, (k + 1) % 2)
        else:
            wait_idx(t + 1, (k + 1) % 3)
            start_gather(t + 1, (k + 1) % 2)

        wait_gather(t, r)
        start_scatter(r, d)

        if k >= 4:
            @pl.when(t + 2 < _NF)
            def _():
                start_idx(t + 2, (k + 2) % 3)
        else:
            start_idx(t + 2, (k + 2) % 3)

    start_idx(0, 0)
    start_idx(1, 1)
    wait_idx(0, 0)
    start_gather(0, 0)

    def six(i, carry):
        t0 = 6 * i
        for k in range(6):
            step(t0 + k, k)
        return carry

    lax.fori_loop(0, _NF // 6, six, 0)
    wait_scatter(1, 2)

    tb = base0 + _NF * _K
    pltpu.sync_copy(edge_hbm.at[pl.ds(_E + tb, _TAIL)], dstT)
    pltpu.async_copy(x_hbm.at[src_all.at[pl.ds(_NF * _K, _TAIL)]],
                     rows0.at[pl.ds(0, _TAIL)], sg0).wait()
    pltpu.sync_copy(rows0.at[pl.ds(0, _TAIL)], acc.at[dstT], add=True)
    pltpu.sync_copy(ones_v.at[pl.ds(0, _TAIL)], cntacc.at[dstT], add=True)

    plsc.subcore_barrier()

    @pl.when(s < 15)
    def _():
        for j in range(4):
            pltpu.sync_copy(acc.at[pl.ds(s * _R632 + j * _K, _K)], rows0)
            pltpu.sync_copy(rows0, psum_hbm.at[c, pl.ds(s * _R632 + j * _K, _K)])
        pltpu.sync_copy(acc.at[pl.ds(s * _R632 + 4 * _K, 120)],
                        rows0.at[pl.ds(0, 120)])
        pltpu.sync_copy(rows0.at[pl.ds(0, 120)],
                        psum_hbm.at[c, pl.ds(s * _R632 + 4 * _K, 120)])
        pltpu.sync_copy(cntacc.at[pl.ds(s * _C640, _C640)], zcnt)
        pltpu.sync_copy(zcnt, pcnt_hbm.at[c, pl.ds(s * _C640, _C640)])

    @pl.when(s == 15)
    def _():
        for j in range(4):
            pltpu.sync_copy(acc.at[pl.ds(15 * _R632 + j * _K, _K)], rows0)
            pltpu.sync_copy(rows0, psum_hbm.at[c, pl.ds(15 * _R632 + j * _K, _K)])
        pltpu.sync_copy(acc.at[pl.ds(15 * _R632 + 4 * _K, 8)],
                        rows0.at[pl.ds(0, 8)])
        pltpu.sync_copy(rows0.at[pl.ds(0, 8)],
                        psum_hbm.at[c, pl.ds(15 * _R632 + 4 * _K, 8)])
        pltpu.sync_copy(cntacc.at[pl.ds(15 * _C640, _CLAST)],
                        zcnt.at[pl.ds(0, _CLAST)])
        pltpu.sync_copy(zcnt.at[pl.ds(0, _CLAST)],
                        pcnt_hbm.at[c, pl.ds(15 * _C640, _CLAST)])


def _sc_segment_sum(x, edge_index):
    mesh = plsc.VectorSubcoreMesh(core_axis_name="c", subcore_axis_name="s")
    z2 = jnp.zeros((_N, _D), jnp.float32)
    z1 = jnp.zeros((_NP,), jnp.float32)
    run = pl.kernel(
        _sc_body,
        out_type=(
            jax.ShapeDtypeStruct((_NC, _N, _D), jnp.float32),
            jax.ShapeDtypeStruct((_NC, _NP), jnp.float32),
        ),
        mesh=mesh,
        scratch_types=[
            pltpu.VMEM((_EPW,), jnp.int32),
            pltpu.VMEM((_K,), jnp.int32),
            pltpu.VMEM((_K,), jnp.int32),
            pltpu.VMEM((_K,), jnp.int32),
            pltpu.VMEM((_K, _D), jnp.float32),
            pltpu.VMEM((_K, _D), jnp.float32),
            pltpu.VMEM((_TAIL,), jnp.int32),
            pltpu.VMEM((_K,), jnp.float32),
            pltpu.VMEM((_C640,), jnp.float32),
            pltpu.VMEM_SHARED((_N, _D), jnp.float32),
            pltpu.VMEM_SHARED((_NP,), jnp.float32),
            pltpu.SemaphoreType.DMA,
            pltpu.SemaphoreType.DMA,
            pltpu.SemaphoreType.DMA,
            pltpu.SemaphoreType.DMA,
            pltpu.SemaphoreType.DMA,
            pltpu.SemaphoreType.DMA,
            pltpu.SemaphoreType.DMA,
        ],
    )
    return run(x, edge_index.reshape(-1), z2, z1)


_R = 2000


def _tc_body(psum_ref, pcnt_ref, x_ref, wl_ref, bl_ref, wr_ref, w1_ref, w2_ref,
             out1_ref, out2_ref, x1_ref):
    summed = psum_ref[0] + psum_ref[1]
    cnt = pcnt_ref[:, 0:1] + pcnt_ref[:, 1:2]
    mean = summed / jnp.maximum(cnt, 1.0)
    x_blk = x_ref[...]
    x1 = (jnp.dot(mean, wl_ref[...].T, preferred_element_type=jnp.float32,
                  precision=lax.Precision.HIGHEST)
          + bl_ref[...]
          + jnp.dot(x_blk, wr_ref[...].T, preferred_element_type=jnp.float32,
                    precision=lax.Precision.HIGHEST))
    x1_ref[...] = x1
    norm = jnp.sqrt(jnp.sum(x1 * x1, axis=1, keepdims=True))
    xn = x1 / jnp.maximum(norm, 1e-12)
    w1 = w1_ref[...]
    w1 = w1 / jnp.maximum(jnp.sqrt(jnp.sum(w1 * w1, axis=0, keepdims=True)), 1e-12)
    out1_ref[...] = 10.0 * jnp.dot(xn, w1, preferred_element_type=jnp.float32,
                                   precision=lax.Precision.HIGHEST)
    w2 = w2_ref[...]
    w2 = w2 / jnp.maximum(jnp.sqrt(jnp.sum(w2 * w2, axis=0, keepdims=True)), 1e-12)
    out2_ref[...] = 10.0 * jnp.dot(xn, w2, preferred_element_type=jnp.float32,
                                   precision=lax.Precision.HIGHEST)


def _tc_heads(psum, pcnt_t, x, W_l, b_l, W_r, W1, W2):
    c1 = W1.shape[1]
    c2 = W2.shape[1]
    grid = _N // _R
    return pl.pallas_call(
        _tc_body,
        grid=(grid,),
        in_specs=[
            pl.BlockSpec((_NC, _R, _D), lambda i: (0, i, 0)),
            pl.BlockSpec((_R, _NC), lambda i: (i, 0)),
            pl.BlockSpec((_R, _D), lambda i: (i, 0)),
            pl.BlockSpec((_D, _D), lambda i: (0, 0)),
            pl.BlockSpec((1, _D), lambda i: (0, 0)),
            pl.BlockSpec((_D, _D), lambda i: (0, 0)),
            pl.BlockSpec((_D, c1), lambda i: (0, 0)),
            pl.BlockSpec((_D, c2), lambda i: (0, 0)),
        ],
        out_specs=[
            pl.BlockSpec((_R, c1), lambda i: (i, 0)),
            pl.BlockSpec((_R, c2), lambda i: (i, 0)),
            pl.BlockSpec((_R, _D), lambda i: (i, 0)),
        ],
        out_shape=[
            jax.ShapeDtypeStruct((_N, c1), jnp.float32),
            jax.ShapeDtypeStruct((_N, c2), jnp.float32),
            jax.ShapeDtypeStruct((_N, _D), jnp.float32),
        ],
    )(psum, pcnt_t, x, W_l, b_l, W_r, W1, W2)


@jax.jit
def kernel(x, edge_index, W_l, b_l, W_r, W1, W2):
    psum, pcnt = _sc_segment_sum(x, edge_index)
    out1, out2, x1 = _tc_heads(psum, pcnt[:, :_N].T, x, W_l, b_l[None, :], W_r,
                               W1, W2)
    return (out1, out2, x1)


# speedup vs baseline: 1.3884x; 1.0592x over previous
import jax
import jax.numpy as jnp
from jax import lax
from jax.experimental import pallas as pl
from jax.experimental.pallas import tpu as pltpu
from jax.experimental.pallas import tpu_sc as plsc

_N = 10000
_E = 320000
_D = 128
_NC = 2
_NS = 16
_NW = _NC * _NS
_EPW = _E // _NW
_K = 128
_NF = 78
_TAIL = _EPW - _NF * _K
_R632 = 632
_RLAST = _N - 15 * _R632
_NP = 10112
_C640 = 640
_CLAST = _NP - 15 * _C640


def _sc_body(x_hbm, edge_hbm, z2_hbm, z1_hbm, psum_hbm, pcnt_hbm,
             src_all, dst0, dst1, dst2, rows0, rows1, dstT, ones_v, zcnt,
             acc, cntacc,
             si0, si1, si2, sg0, sg1, ss0, ss1):
    c = lax.axis_index("c")
    s = lax.axis_index("s")
    wid = s * _NC + c
    base0 = wid * _EPW

    pltpu.async_copy(edge_hbm.at[pl.ds(base0, _EPW)], src_all, si2)

    pltpu.sync_copy(z2_hbm.at[pl.ds(0, _K)], rows0)
    pltpu.sync_copy(z1_hbm.at[pl.ds(0, _C640)], zcnt)

    @pl.when(s < 15)
    def _():
        for j in range(4):
            pltpu.sync_copy(rows0, acc.at[pl.ds(s * _R632 + j * _K, _K)])
        pltpu.sync_copy(rows0.at[pl.ds(0, 120)],
                        acc.at[pl.ds(s * _R632 + 4 * _K, 120)])
        pltpu.sync_copy(zcnt, cntacc.at[pl.ds(s * _C640, _C640)])

    @pl.when(s == 15)
    def _():
        for j in range(4):
            pltpu.sync_copy(rows0, acc.at[pl.ds(15 * _R632 + j * _K, _K)])
        pltpu.sync_copy(rows0.at[pl.ds(0, 8)],
                        acc.at[pl.ds(15 * _R632 + 4 * _K, 8)])
        pltpu.sync_copy(zcnt.at[pl.ds(0, _CLAST)],
                        cntacc.at[pl.ds(15 * _C640, _CLAST)])

    for i in range(_K // 16):
        ones_v[pl.ds(i * 16, 16)] = jnp.ones((16,), jnp.float32)

    pltpu.make_async_copy(edge_hbm.at[pl.ds(base0, _EPW)], src_all, si2).wait()

    plsc.subcore_barrier()

    dsts = (dst0, dst1, dst2)
    isems = (si0, si1, si2)
    rows = (rows0, rows1)
    gsems = (sg0, sg1)
    ssems = (ss0, ss1)

    def start_idx(t, d):
        pltpu.async_copy(edge_hbm.at[pl.ds(_E + base0 + t * _K, _K)],
                         dsts[d], isems[d])

    def wait_idx(t, d):
        pltpu.make_async_copy(edge_hbm.at[pl.ds(_E + base0 + t * _K, _K)],
                              dsts[d], isems[d]).wait()

    def start_gather(t, r):
        pltpu.async_copy(x_hbm.at[src_all.at[pl.ds(t * _K, _K)]],
                         rows[r], gsems[r])

    def wait_gather(t, r):
        pltpu.make_async_copy(x_hbm.at[src_all.at[pl.ds(t * _K, _K)]],
                              rows[r], gsems[r]).wait()

    def start_scatter(r, d):
        pltpu.async_copy(rows[r], acc.at[dsts[d]], ssems[r], add=True)
        pltpu.async_copy(ones_v, cntacc.at[dsts[d]], ssems[r], add=True)

    def wait_scatter(r, d):
        pltpu.make_async_copy(rows[r], acc.at[dsts[d]], ssems[r]).wait()
        pltpu.make_async_copy(ones_v, cntacc.at[dsts[d]], ssems[r]).wait()

    def step(t, k):
        r = k % 2
        d = k % 3
        if k == 0:
            @pl.when(t > 0)
            def _():
                wait_scatter(1, 2)
        else:
            wait_scatter((k - 1) % 2, (k - 1) % 3)

        if k == 5:
            @pl.when(t + 1 < _NF)
            def _():
                wait_idx(t + 1, (k + 1) % 3)
                start_gather(t + 1, (k + 1) % 2)
        else:
            wait_idx(t + 1, (k + 1) % 3)
            start_gather(t + 1, (k + 1) % 2)

        wait_gather(t, r)
        start_scatter(r, d)

        if k >= 4:
            @pl.when(t + 2 < _NF)
            def _():
                start_idx(t + 2, (k + 2) % 3)
        else:
            start_idx(t + 2, (k + 2) % 3)

    start_idx(0, 0)
    start_idx(1, 1)
    wait_idx(0, 0)
    start_gather(0, 0)

    def six(i, carry):
        t0 = 6 * i
        for k in range(6):
            step(t0 + k, k)
        return carry

    lax.fori_loop(0, _NF // 6, six, 0)
    wait_scatter(1, 2)

    tb = base0 + _NF * _K
    pltpu.sync_copy(edge_hbm.at[pl.ds(_E + tb, _TAIL)], dstT)
    pltpu.async_copy(x_hbm.at[src_all.at[pl.ds(_NF * _K, _TAIL)]],
                     rows0.at[pl.ds(0, _TAIL)], sg0).wait()
    pltpu.sync_copy(rows0.at[pl.ds(0, _TAIL)], acc.at[dstT], add=True)
    pltpu.sync_copy(ones_v.at[pl.ds(0, _TAIL)], cntacc.at[dstT], add=True)

    plsc.subcore_barrier()

    @pl.when(s < 15)
    def _():
        for j in range(4):
            pltpu.sync_copy(acc.at[pl.ds(s * _R632 + j * _K, _K)], rows0)
            pltpu.sync_copy(rows0, psum_hbm.at[c, pl.ds(s * _R632 + j * _K, _K)])
        pltpu.sync_copy(acc.at[pl.ds(s * _R632 + 4 * _K, 120)],
                        rows0.at[pl.ds(0, 120)])
        pltpu.sync_copy(rows0.at[pl.ds(0, 120)],
                        psum_hbm.at[c, pl.ds(s * _R632 + 4 * _K, 120)])
        pltpu.sync_copy(cntacc.at[pl.ds(s * _C640, _C640)], zcnt)
        pltpu.sync_copy(zcnt, pcnt_hbm.at[c, pl.ds(s * _C640, _C640)])

    @pl.when(s == 15)
    def _():
        for j in range(4):
            pltpu.sync_copy(acc.at[pl.ds(15 * _R632 + j * _K, _K)], rows0)
            pltpu.sync_copy(rows0, psum_hbm.at[c, pl.ds(15 * _R632 + j * _K, _K)])
        pltpu.sync_copy(acc.at[pl.ds(15 * _R632 + 4 * _K, 8)],
                        rows0.at[pl.ds(0, 8)])
        pltpu.sync_copy(rows0.at[pl.ds(0, 8)],
                        psum_hbm.at[c, pl.ds(15 * _R632 + 4 * _K, 8)])
        pltpu.sync_copy(cntacc.at[pl.ds(15 * _C640, _CLAST)],
                        zcnt.at[pl.ds(0, _CLAST)])
        pltpu.sync_copy(zcnt.at[pl.ds(0, _CLAST)],
                        pcnt_hbm.at[c, pl.ds(15 * _C640, _CLAST)])


def _sc_segment_sum(x, edge_index):
    mesh = plsc.VectorSubcoreMesh(core_axis_name="c", subcore_axis_name="s")
    z2 = jnp.zeros((_N, _D), jnp.float32)
    z1 = jnp.zeros((_NP,), jnp.float32)
    run = pl.kernel(
        _sc_body,
        out_type=(
            jax.ShapeDtypeStruct((_NC, _N, _D), jnp.float32),
            jax.ShapeDtypeStruct((_NC, _NP), jnp.float32),
        ),
        mesh=mesh,
        scratch_types=[
            pltpu.VMEM((_EPW,), jnp.int32),
            pltpu.VMEM((_K,), jnp.int32),
            pltpu.VMEM((_K,), jnp.int32),
            pltpu.VMEM((_K,), jnp.int32),
            pltpu.VMEM((_K, _D), jnp.float32),
            pltpu.VMEM((_K, _D), jnp.float32),
            pltpu.VMEM((_TAIL,), jnp.int32),
            pltpu.VMEM((_K,), jnp.float32),
            pltpu.VMEM((_C640,), jnp.float32),
            pltpu.VMEM_SHARED((_N, _D), jnp.float32),
            pltpu.VMEM_SHARED((_NP,), jnp.float32),
            pltpu.SemaphoreType.DMA,
            pltpu.SemaphoreType.DMA,
            pltpu.SemaphoreType.DMA,
            pltpu.SemaphoreType.DMA,
            pltpu.SemaphoreType.DMA,
            pltpu.SemaphoreType.DMA,
            pltpu.SemaphoreType.DMA,
        ],
    )
    return run(x, edge_index.reshape(-1), z2, z1)


_R = 2000


def _tc_pre_body(x_ref, wr_ref, bl_ref, w1_ref, w2_ref,
                 y_ref, w1n_ref, w2n_ref):
    y_ref[...] = (jnp.dot(x_ref[...], wr_ref[...].T,
                          preferred_element_type=jnp.float32,
                          precision=lax.Precision.HIGHEST)
                  + bl_ref[...])
    w1 = w1_ref[...]
    w1n_ref[...] = w1 / jnp.maximum(
        jnp.sqrt(jnp.sum(w1 * w1, axis=0, keepdims=True)), 1e-12)
    w2 = w2_ref[...]
    w2n_ref[...] = w2 / jnp.maximum(
        jnp.sqrt(jnp.sum(w2 * w2, axis=0, keepdims=True)), 1e-12)


def _tc_pre(x, W_r, b_l, W1, W2):
    c1 = W1.shape[1]
    c2 = W2.shape[1]
    return pl.pallas_call(
        _tc_pre_body,
        grid=(_N // _R,),
        in_specs=[
            pl.BlockSpec((_R, _D), lambda i: (i, 0)),
            pl.BlockSpec((_D, _D), lambda i: (0, 0)),
            pl.BlockSpec((1, _D), lambda i: (0, 0)),
            pl.BlockSpec((_D, c1), lambda i: (0, 0)),
            pl.BlockSpec((_D, c2), lambda i: (0, 0)),
        ],
        out_specs=[
            pl.BlockSpec((_R, _D), lambda i: (i, 0)),
            pl.BlockSpec((_D, c1), lambda i: (0, 0)),
            pl.BlockSpec((_D, c2), lambda i: (0, 0)),
        ],
        out_shape=[
            jax.ShapeDtypeStruct((_N, _D), jnp.float32),
            jax.ShapeDtypeStruct((_D, c1), jnp.float32),
            jax.ShapeDtypeStruct((_D, c2), jnp.float32),
        ],
    )(x, W_r, b_l, W1, W2)


def _tc_post_body(psum_ref, pcnt_ref, y_ref, wl_ref, w1n_ref, w2n_ref,
                  out1_ref, out2_ref, x1_ref):
    summed = psum_ref[0] + psum_ref[1]
    cnt = pcnt_ref[:, 0:1] + pcnt_ref[:, 1:2]
    mean = summed / jnp.maximum(cnt, 1.0)
    x1 = (jnp.dot(mean, wl_ref[...].T, preferred_element_type=jnp.float32,
                  precision=lax.Precision.HIGHEST)
          + y_ref[...])
    x1_ref[...] = x1
    norm = jnp.sqrt(jnp.sum(x1 * x1, axis=1, keepdims=True))
    xn = x1 / jnp.maximum(norm, 1e-12)
    out1_ref[...] = 10.0 * jnp.dot(xn, w1n_ref[...],
                                   preferred_element_type=jnp.float32,
                                   precision=lax.Precision.HIGHEST)
    out2_ref[...] = 10.0 * jnp.dot(xn, w2n_ref[...],
                                   preferred_element_type=jnp.float32,
                                   precision=lax.Precision.HIGHEST)


def _tc_post(psum, pcnt_t, y, W_l, W1n, W2n):
    c1 = W1n.shape[1]
    c2 = W2n.shape[1]
    return pl.pallas_call(
        _tc_post_body,
        grid=(_N // _R,),
        in_specs=[
            pl.BlockSpec((_NC, _R, _D), lambda i: (0, i, 0)),
            pl.BlockSpec((_R, _NC), lambda i: (i, 0)),
            pl.BlockSpec((_R, _D), lambda i: (i, 0)),
            pl.BlockSpec((_D, _D), lambda i: (0, 0)),
            pl.BlockSpec((_D, c1), lambda i: (0, 0)),
            pl.BlockSpec((_D, c2), lambda i: (0, 0)),
        ],
        out_specs=[
            pl.BlockSpec((_R, c1), lambda i: (i, 0)),
            pl.BlockSpec((_R, c2), lambda i: (i, 0)),
            pl.BlockSpec((_R, _D), lambda i: (i, 0)),
        ],
        out_shape=[
            jax.ShapeDtypeStruct((_N, c1), jnp.float32),
            jax.ShapeDtypeStruct((_N, c2), jnp.float32),
            jax.ShapeDtypeStruct((_N, _D), jnp.float32),
        ],
    )(psum, pcnt_t, y, W_l, W1n, W2n)


@jax.jit
def kernel(x, edge_index, W_l, b_l, W_r, W1, W2):
    psum, pcnt = _sc_segment_sum(x, edge_index)
    y, W1n, W2n = _tc_pre(x, W_r, b_l[None, :], W1, W2)
    out1, out2, x1 = _tc_post(psum, pcnt[:, :_N].T, y, W_l, W1n, W2n)
    return (out1, out2, x1)
